# baseline (speedup 1.0000x reference)
import functools

import jax
import jax.numpy as jnp
from jax import lax
from jax.experimental import pallas as pl
from jax.experimental.pallas import tpu as pltpu
from jax.experimental.pallas import tpu_sc as plsc

_NC = 2
_NS = 16
_NW = _NC * _NS
_LANES = 16


def kernel(txt, token_table, pos_table):
    B, L = txt.shape
    V, D = token_table.shape
    DS = D // 128
    NP = B // 16
    PPW = NP // _NW
    assert D % 128 == 0 and B % (16 * _NW) == 0
    NIDX = 16 * DS
    NVEC = NIDX // _LANES
    GDEEP = 8
    ODEEP = 4
    assert PPW % GDEEP == 0 and GDEEP % ODEEP == 0
    K2 = L * PPW // GDEEP

    mesh = plsc.VectorSubcoreMesh(core_axis_name="c", subcore_axis_name="s")

    @functools.partial(
        pl.kernel,
        out_type=jax.ShapeDtypeStruct((L, NP, NIDX, 128), jnp.float32),
        mesh=mesh,
        compiler_params=pltpu.CompilerParams(
            use_tc_tiling_on_sc=False, needs_layout_passes=False),
        scratch_types=[
            pltpu.VMEM((16 * PPW,), jnp.int32),
            pltpu.VMEM((D,), jnp.float32),
            [pltpu.VMEM((NIDX,), jnp.int32) for _ in range(GDEEP)],
            [pltpu.VMEM((NIDX, 128), jnp.bfloat16) for _ in range(GDEEP)],
            [pltpu.VMEM((NIDX, 128), jnp.float32) for _ in range(ODEEP)],
            [pltpu.SemaphoreType.DMA for _ in range(GDEEP)],
            [pltpu.SemaphoreType.DMA for _ in range(ODEEP)],
        ],
    )
    def run(txtT_hbm, tokT_hbm, pos_hbm, out_hbm,
            txt_v, pos_v, idx_bufs, gbufs, obufs, sems_in, sems_out):
        wid = lax.axis_index("s") * _NC + lax.axis_index("c")

        def stage_l(l):
            pltpu.sync_copy(txtT_hbm.at[l, pl.ds(16 * PPW * wid, 16 * PPW)], txt_v)
            pltpu.sync_copy(pos_hbm.at[l], pos_v)

        def build_idx(q, j):
            iot = lax.iota(jnp.int32, _LANES)
            lo = iot & 7
            hi = iot >> 3
            for c in range(NVEC):
                g = 16 * j + 8 * (c // 3) + lo
                vals = plsc.load_gather(txt_v, [g])
                idx_bufs[q][pl.ds(16 * c, 16)] = vals * DS + (hi + 2 * (c % 3))

        def gather(q):
            pltpu.async_copy(tokT_hbm.at[idx_bufs[q]], gbufs[q], sems_in[q])

        def gather_wait(q):
            pltpu.make_async_copy(tokT_hbm.at[idx_bufs[q]], gbufs[q], sems_in[q]).wait()

        def add_pos(q):
            o = q % ODEEP

            def dt_body(dt, carry):
                for jj in range(4):
                    pa = pos_v[pl.ds(dt * 128 + 32 * jj, 16)]
                    pb = pos_v[pl.ds(dt * 128 + 32 * jj + 16, 16)]
                    for t in range(2):
                        base = t * (8 * DS) + dt * 8
                        for br in range(8):
                            row = base + br
                            x = gbufs[q][row, pl.ds(32 * jj, 32)]
                            a, b = plsc.unpack(
                                x, format=plsc.PackFormat.INTERLEAVED)
                            obufs[o][row, pl.ds(32 * jj, 16)] = a + pa
                            obufs[o][row, pl.ds(32 * jj + 16, 16)] = b + pb
                return carry

            lax.fori_loop(0, DS, dt_body, 0)

        def scatter(q, l, pt):
            o = q % ODEEP
            pltpu.async_copy(obufs[o], out_hbm.at[l, pt], sems_out[o])

        def scatter_wait(q, l):
            o = q % ODEEP
            pltpu.make_async_copy(obufs[o], out_hbm.at[l, 0], sems_out[o]).wait()

        stage_l(0)
        for q in range(GDEEP):
            build_idx(q, q)
            gather(q)

        def body(k, carry):
            m = k % (PPW // GDEEP)
            l = k // (PPW // GDEEP)
            not_last = k < K2 - 1

            for q in range(GDEEP):
                gather_wait(q)
                if q >= ODEEP:
                    scatter_wait(q, l)
                else:
                    @pl.when(k > 0)
                    def _():
                        scatter_wait(q, l)
                add_pos(q)
                scatter(q, l, PPW * wid + GDEEP * m + q)

            @pl.when((m == PPW // GDEEP - 1) & not_last)
            def _():
                stage_l(l + 1)

            for q in range(GDEEP):

                @pl.when(not_last)
                def _():
                    jn = (GDEEP * (m + 1) + q) % PPW
                    build_idx(q, jn)
                    gather(q)

            return carry

        lax.fori_loop(0, K2, body, 0)

        for q in range(ODEEP):
            scatter_wait(q, 0)

    txtT = txt.T
    tokT = (token_table.astype(jnp.bfloat16)
            .reshape(-1, 2, 16).swapaxes(-1, -2).reshape(V * DS, 128))
    out5 = run(txtT, tokT, pos_table)
    return (out5.reshape(L, NP, 2, DS, 8, 128)
                .transpose(1, 2, 4, 0, 3, 5)
                .reshape(B, L, D))


# speedup vs baseline: 2.7818x; 2.7818x over previous
import functools

import jax
import jax.numpy as jnp
from jax import lax
from jax.experimental import pallas as pl
from jax.experimental.pallas import tpu as pltpu
from jax.experimental.pallas import tpu_sc as plsc

_NC = 2
_NS = 16
_NW = _NC * _NS
_LANES = 16


def kernel(txt, token_table, pos_table):
    B, L = txt.shape
    V, D = token_table.shape
    DS = D // 128
    NP = B // 16
    PPW = NP // _NW
    assert D % 128 == 0 and B % (16 * _NW) == 0
    NIDX = 16 * DS
    NVEC = NIDX // _LANES
    UNROLL = 8
    assert PPW % UNROLL == 0
    K2 = L * PPW // UNROLL

    mesh = plsc.VectorSubcoreMesh(core_axis_name="c", subcore_axis_name="s")

    @functools.partial(
        pl.kernel,
        out_type=jax.ShapeDtypeStruct((L, NP, NIDX, 128), jnp.float32),
        mesh=mesh,
        compiler_params=pltpu.CompilerParams(
            use_tc_tiling_on_sc=False, needs_layout_passes=False),
        scratch_types=[
            pltpu.VMEM((16 * PPW,), jnp.int32),
            pltpu.VMEM((D,), jnp.float32),
            [pltpu.VMEM((NIDX,), jnp.int32) for _ in range(UNROLL)],
            [pltpu.VMEM((NIDX, 128), jnp.float32) for _ in range(UNROLL)],
            [pltpu.SemaphoreType.DMA for _ in range(UNROLL)],
            [pltpu.SemaphoreType.DMA for _ in range(UNROLL)],
        ],
    )
    def run(txtT_hbm, tokT_hbm, pos_hbm, out_hbm,
            txt_v, pos_v, idx_bufs, gbufs, sems_in, sems_out):
        wid = lax.axis_index("s") * _NC + lax.axis_index("c")

        def stage_l(l):
            pltpu.sync_copy(txtT_hbm.at[l, pl.ds(16 * PPW * wid, 16 * PPW)], txt_v)
            pltpu.sync_copy(pos_hbm.at[l], pos_v)

        def build_idx(q, j):
            iot = lax.iota(jnp.int32, _LANES)
            lo = iot & 7
            hi = iot >> 3
            for c in range(NVEC):
                g = 16 * j + 8 * (c // 3) + lo
                vals = plsc.load_gather(txt_v, [g])
                idx_bufs[q][pl.ds(16 * c, 16)] = vals * DS + (hi + 2 * (c % 3))

        def gather(q):
            return pltpu.async_copy(tokT_hbm.at[idx_bufs[q]], gbufs[q], sems_in[q])

        def gather_wait(q):
            pltpu.make_async_copy(tokT_hbm.at[idx_bufs[q]], gbufs[q], sems_in[q]).wait()

        def add_pos(q):
            def dt_body(dt, carry):
                for jj in range(8):
                    v = pos_v[pl.ds(dt * 128 + 16 * jj, 16)]
                    for t in range(2):
                        row = t * (8 * DS) + dt * 8
                        for br in range(8):
                            plsc.addupdate(
                                gbufs[q].at[row + br, pl.ds(16 * jj, 16)], v)
                return carry

            lax.fori_loop(0, DS, dt_body, 0)

        def scatter(q, l, pt):
            return pltpu.async_copy(gbufs[q], out_hbm.at[l, pt], sems_out[q])

        stage_l(0)
        for q in range(UNROLL):
            build_idx(q, q)
            gather(q)

        def body(k, carry):
            m = k % (PPW // UNROLL)
            l = k // (PPW // UNROLL)
            not_last = k < K2 - 1

            sc = []
            for q in range(UNROLL):
                gather_wait(q)
                add_pos(q)
                sc.append(scatter(q, l, PPW * wid + UNROLL * m + q))

            @pl.when((m == PPW // UNROLL - 1) & not_last)
            def _():
                stage_l(l + 1)

            for q in range(UNROLL):
                sc[q].wait()

                @pl.when(not_last)
                def _():
                    jn = (UNROLL * (m + 1) + q) % PPW
                    build_idx(q, jn)
                    gather(q)

            return carry

        lax.fori_loop(0, K2, body, 0)

    txtT = txt.T
    tokT = token_table.reshape(V * DS, 128)
    out5 = run(txtT, tokT, pos_table)
    return (out5.reshape(L, NP, 2, DS, 8, 128)
                .transpose(1, 2, 4, 0, 3, 5)
                .reshape(B, L, D))


# speedup vs baseline: 2.8560x; 1.0267x over previous
import functools

import jax
import jax.numpy as jnp
from jax import lax
from jax.experimental import pallas as pl
from jax.experimental.pallas import tpu as pltpu
from jax.experimental.pallas import tpu_sc as plsc

_NC = 2
_NS = 16
_NW = _NC * _NS
_LANES = 16


def kernel(txt, token_table, pos_table):
    B, L = txt.shape
    V, D = token_table.shape
    DS = D // 128
    NP = B // 16
    PPW = NP // _NW
    assert D % 128 == 0 and B % (16 * _NW) == 0
    NIDX = 16 * DS
    NVEC = NIDX // _LANES
    UNROLL = 8
    assert PPW % UNROLL == 0
    K2 = L * PPW // UNROLL

    mesh = plsc.VectorSubcoreMesh(core_axis_name="c", subcore_axis_name="s")

    @functools.partial(
        pl.kernel,
        out_type=jax.ShapeDtypeStruct((L, NP, NIDX, 128), jnp.float32),
        mesh=mesh,
        compiler_params=pltpu.CompilerParams(
            use_tc_tiling_on_sc=False, needs_layout_passes=False),
        scratch_types=[
            pltpu.VMEM((2, 16 * PPW), jnp.int32),
            pltpu.VMEM((2, D), jnp.float32),
            [pltpu.VMEM((NIDX,), jnp.int32) for _ in range(UNROLL)],
            [pltpu.VMEM((NIDX, 128), jnp.float32) for _ in range(UNROLL)],
            [pltpu.SemaphoreType.DMA for _ in range(UNROLL)],
            [pltpu.SemaphoreType.DMA for _ in range(UNROLL)],
        ],
    )
    def run(txtT_hbm, tokT_hbm, pos_hbm, out_hbm,
            txt_v, pos_v, idx_bufs, gbufs, sems_in, sems_out):
        wid = lax.axis_index("s") * _NC + lax.axis_index("c")

        def stage_pair(l):
            pltpu.sync_copy(
                txtT_hbm.at[pl.ds(l, 2), pl.ds(16 * PPW * wid, 16 * PPW)], txt_v)
            pltpu.sync_copy(pos_hbm.at[pl.ds(l, 2)], pos_v)

        def build_idx(q, j, lpar):
            iot = lax.iota(jnp.int32, _LANES)
            lo = iot & 7
            hi = iot >> 3
            row = (iot & 0) + lpar
            for c in range(NVEC):
                g = 16 * j + 8 * (c // 3) + lo
                vals = plsc.load_gather(txt_v, [row, g])
                idx_bufs[q][pl.ds(16 * c, 16)] = vals * DS + (hi + 2 * (c % 3))

        def gather(q):
            return pltpu.async_copy(tokT_hbm.at[idx_bufs[q]], gbufs[q], sems_in[q])

        def gather_wait(q):
            pltpu.make_async_copy(tokT_hbm.at[idx_bufs[q]], gbufs[q], sems_in[q]).wait()

        def add_pos(q, lpar):
            def dt_body(dt, carry):
                for jj in range(8):
                    v = pos_v[lpar, pl.ds(dt * 128 + 16 * jj, 16)]
                    for t in range(2):
                        row = t * (8 * DS) + dt * 8
                        for br in range(8):
                            plsc.addupdate(
                                gbufs[q].at[row + br, pl.ds(16 * jj, 16)], v)
                return carry

            lax.fori_loop(0, DS, dt_body, 0)

        def scatter(q, l, pt):
            return pltpu.async_copy(gbufs[q], out_hbm.at[l, pt], sems_out[q])

        stage_pair(0)
        for q in range(UNROLL):
            build_idx(q, q, 0)
            gather(q)

        def body(k, carry):
            m = k % (PPW // UNROLL)
            l = k // (PPW // UNROLL)
            lpar = l & 1
            not_last = k < K2 - 1
            crossing = (m == PPW // UNROLL - 1) & not_last

            sc = []
            for q in range(UNROLL):
                gather_wait(q)
                add_pos(q, lpar)
                sc.append(scatter(q, l, PPW * wid + UNROLL * m + q))

            @pl.when(crossing & (lpar == 1))
            def _():
                stage_pair(l + 1)

            for q in range(UNROLL):
                sc[q].wait()

                @pl.when(not_last)
                def _():
                    jn = (UNROLL * (m + 1) + q) % PPW
                    lpn = jnp.where(m == PPW // UNROLL - 1, 1 - lpar, lpar)
                    build_idx(q, jn, lpn)
                    gather(q)

            return carry

        lax.fori_loop(0, K2, body, 0)

    txtT = txt.T
    tokT = token_table.reshape(V * DS, 128)
    out5 = run(txtT, tokT, pos_table)
    return (out5.reshape(L, NP, 2, DS, 8, 128)
                .transpose(1, 2, 4, 0, 3, 5)
                .reshape(B, L, D))
